# Initial kernel scaffold; baseline (speedup 1.0000x reference)
import jax
import jax.numpy as jnp
from jax.experimental import pallas as pl


def kernel(x, edge_index, edge_weight, batch, W1, b1, W2, b2, W3, b3, fcW, fcb):
    raise NotImplementedError("write your pallas kernel here")


# speedup vs baseline: 9.8305x; 9.8305x over previous
import functools
import jax
import jax.numpy as jnp
from jax import lax
from jax.experimental import pallas as pl
from jax.experimental.pallas import tpu as pltpu
from jax.experimental.pallas import tpu_sc as plsc

N = 10000
E = 320000
F = 128
G = 64

R = 80
NROWS = E // R
ROWS_T = NROWS // 32
NTILE = N // 16
NBUF = 5
LEAD = 3

_mesh = plsc.VectorSubcoreMesh(core_axis_name="c", subcore_axis_name="s")
_sc_params = pltpu.CompilerParams(use_tc_tiling_on_sc=False,
                                  needs_layout_passes=False)


def _rsqrt16(d):
    db = plsc.bitcast(d, jnp.int32)
    yb = jnp.int32(0x5F3759DF) - lax.shift_right_logical(db, 1)
    y = plsc.bitcast(yb, jnp.float32)
    for _ in range(3):
        y = y * (1.5 - 0.5 * ((d * y) * y))
    return jnp.where(d > 0, y, 0.0)


def _prep(srcR, dstR, wR, zN):
    def body(srcR_h, dstR_h, wR_h, zN_h, normR_h,
             sbuf, dbuf, wbuf, wmbuf, tmp, tail16, dis_full, nbuf, deg, dsem):
        cid = lax.axis_index("c")
        sid = lax.axis_index("s")
        pltpu.sync_copy(zN_h.at[pl.ds(sid * 624, 624)],
                        deg.at[pl.ds(sid * 624, 624)])

        @pl.when(sid == 0)
        def _():
            pltpu.sync_copy(zN_h.at[pl.ds(9984, 16)], deg.at[pl.ds(9984, 16)])

        plsc.subcore_barrier()

        @pl.loop(0, 10)
        def _(ch):
            r0 = sid * (NROWS // 16) + ch * 25
            pltpu.sync_copy(srcR_h.at[pl.ds(r0, 25)], sbuf)
            pltpu.sync_copy(dstR_h.at[pl.ds(r0, 25)], dbuf)
            pltpu.sync_copy(wR_h.at[pl.ds(r0, 25)], wbuf)
            for r in range(25):
                for j in range(R // 16):
                    sl = (r, pl.ds(j * 16, 16))
                    wmbuf[sl] = jnp.where(sbuf[sl] == dbuf[sl], 0.0, wbuf[sl])
            for r in range(25):
                pltpu.async_copy(wmbuf.at[r], deg.at[sbuf.at[r]], dsem, add=True)
            for r in range(25):
                pltpu.make_async_copy(wmbuf.at[r], deg.at[sbuf.at[r]], dsem).wait()

        plsc.subcore_barrier()

        base = sid * 624
        pltpu.sync_copy(deg.at[pl.ds(base, 624)], tmp)
        for i in range(39):
            o = i * 16
            tmp[pl.ds(o, 16)] = _rsqrt16(tmp[pl.ds(o, 16)])
        pltpu.sync_copy(tmp, deg.at[pl.ds(base, 624)])

        @pl.when(sid == 0)
        def _():
            pltpu.sync_copy(deg.at[pl.ds(9984, 16)], tail16)
            tail16[...] = _rsqrt16(tail16[...])
            pltpu.sync_copy(tail16, deg.at[pl.ds(9984, 16)])

        plsc.subcore_barrier()

        pltpu.sync_copy(deg, dis_full)
        w = cid * 16 + sid

        @pl.loop(0, 5)
        def _(ch):
            r0 = w * ROWS_T + ch * 25
            pltpu.sync_copy(srcR_h.at[pl.ds(r0, 25)], sbuf)
            pltpu.sync_copy(dstR_h.at[pl.ds(r0, 25)], dbuf)
            pltpu.sync_copy(wR_h.at[pl.ds(r0, 25)], wbuf)
            for r in range(25):
                for j in range(R // 16):
                    sl = (r, pl.ds(j * 16, 16))
                    sv = sbuf[sl]
                    dv = dbuf[sl]
                    wm = jnp.where(sv == dv, 0.0, wbuf[sl])
                    dss = plsc.load_gather(dis_full, [sv])
                    dsd = plsc.load_gather(dis_full, [dv])
                    nbuf[sl] = -(dss * wm * dsd)
            pltpu.sync_copy(nbuf, normR_h.at[pl.ds(r0, 25)])

    k = pl.kernel(
        body,
        out_type=jax.ShapeDtypeStruct((NROWS, R), jnp.float32),
        mesh=_mesh,
        compiler_params=_sc_params,
        scratch_types=[
            pltpu.VMEM((25, R), jnp.int32),
            pltpu.VMEM((25, R), jnp.int32),
            pltpu.VMEM((25, R), jnp.float32),
            pltpu.VMEM((25, R), jnp.float32),
            pltpu.VMEM((624,), jnp.float32),
            pltpu.VMEM((16,), jnp.float32),
            pltpu.VMEM((N,), jnp.float32),
            pltpu.VMEM((25, R), jnp.float32),
            pltpu.VMEM_SHARED((N,), jnp.float32),
            pltpu.SemaphoreType.DMA,
        ],
    )
    return k(srcR, dstR, wR, zN)


FH = F // 2


def _spmm(th0, th1, srcR, dstR, normR, zNH):
    def body(th0_h, th1_h, srcR_h, dstR_h, normR_h, zNH_h, out_h,
             src_all, dst_all, norm_all, r0b, r1b, r2b, r3b, r4b,
             acc, gsem, ssem):
        rows = [r0b, r1b, r2b, r3b, r4b]
        cid = lax.axis_index("c")
        sid = lax.axis_index("s")
        w = cid * 16 + sid
        row0 = w * ROWS_T
        pltpu.sync_copy(srcR_h.at[pl.ds(row0, ROWS_T)], src_all)
        pltpu.sync_copy(dstR_h.at[pl.ds(row0, ROWS_T)], dst_all)
        pltpu.sync_copy(normR_h.at[pl.ds(row0, ROWS_T)], norm_all)
        n0 = sid * NTILE

        for fh, table_h in ((0, th0_h), (1, th1_h)):
            pltpu.sync_copy(zNH_h.at[pl.ds(n0, NTILE)], acc.at[pl.ds(n0, NTILE)])
            plsc.subcore_barrier()

            for b in range(LEAD):
                pltpu.async_copy(table_h.at[src_all.at[b]], rows[b], gsem.at[b])

            @pl.loop(0, ROWS_T, step=NBUF)
            def _(i):
                for b in range(NBUF):
                    c = i + b
                    pltpu.make_async_copy(
                        table_h.at[src_all.at[c]], rows[b], gsem.at[b]).wait()

                    @pl.loop(0, R)
                    def _(e):
                        nv = plsc.load_gather(
                            norm_all,
                            [jnp.full((16,), c, jnp.int32),
                             jnp.full((16,), e, jnp.int32)])
                        for j in range(FH // 16):
                            sl = (e, pl.ds(j * 16, 16))
                            rows[b][sl] = rows[b][sl] * nv

                    pltpu.async_copy(rows[b], acc.at[dst_all.at[c]], ssem.at[b],
                                     add=True)
                    b2 = (b - 2) % NBUF

                    @pl.when(c >= 2)
                    def _():
                        pltpu.make_async_copy(
                            rows[b2], acc.at[dst_all.at[c - 2]],
                            ssem.at[b2]).wait()

                    b3 = (b + LEAD) % NBUF

                    @pl.when(c + LEAD < ROWS_T)
                    def _():
                        pltpu.async_copy(
                            table_h.at[src_all.at[c + LEAD]], rows[b3],
                            gsem.at[b3])

            for c in (ROWS_T - 2, ROWS_T - 1):
                b = c % NBUF
                pltpu.make_async_copy(
                    rows[b], acc.at[dst_all.at[c]], ssem.at[b]).wait()
            plsc.subcore_barrier()
            pltpu.sync_copy(acc.at[pl.ds(n0, NTILE)],
                            out_h.at[cid, fh, pl.ds(n0, NTILE)])
            plsc.subcore_barrier()

    k = pl.kernel(
        body,
        out_type=jax.ShapeDtypeStruct((2, 2, N, FH), jnp.float32),
        mesh=_mesh,
        compiler_params=_sc_params,
        scratch_types=[
            pltpu.VMEM((ROWS_T, R), jnp.int32),
            pltpu.VMEM((ROWS_T, R), jnp.int32),
            pltpu.VMEM((ROWS_T, R), jnp.float32),
            pltpu.VMEM((R, FH), jnp.float32),
            pltpu.VMEM((R, FH), jnp.float32),
            pltpu.VMEM((R, FH), jnp.float32),
            pltpu.VMEM((R, FH), jnp.float32),
            pltpu.VMEM((R, FH), jnp.float32),
            pltpu.VMEM_SHARED((N, FH), jnp.float32),
            pltpu.SemaphoreType.DMA((NBUF,)),
            pltpu.SemaphoreType.DMA((NBUF,)),
        ],
    )
    return k(th0, th1, srcR, dstR, normR, zNH)


BN = 1000


def _sum_tc(q):
    def body(q_ref, o0_ref, o1_ref):
        o0_ref[...] = q_ref[0, 0] + q_ref[1, 0]
        o1_ref[...] = q_ref[0, 1] + q_ref[1, 1]

    return pl.pallas_call(
        body,
        grid=(N // BN,),
        in_specs=[pl.BlockSpec((2, 2, BN, FH), lambda i: (0, 0, i, 0))],
        out_specs=[pl.BlockSpec((BN, FH), lambda i: (i, 0))] * 2,
        out_shape=[jax.ShapeDtypeStruct((N, FH), jnp.float32)] * 2,
    )(q)


def _combine_tc(h, t10, t11, r, W, b2d, relu):
    def body(h_ref, t10_ref, t11_ref, r_ref, w_ref, b_ref, o_ref):
        t1 = jnp.concatenate([t10_ref[...], t11_ref[...]], axis=1)
        lt1 = jnp.concatenate([r_ref[0, 0] + r_ref[1, 0],
                               r_ref[0, 1] + r_ref[1, 1]], axis=1)
        tx2 = 2.0 * lt1 - h_ref[...]
        acc = jnp.dot(h_ref[...], w_ref[0], preferred_element_type=jnp.float32)
        acc = acc + jnp.dot(t1, w_ref[1], preferred_element_type=jnp.float32)
        acc = acc + jnp.dot(tx2, w_ref[2], preferred_element_type=jnp.float32)
        acc = acc + b_ref[...]
        o_ref[...] = jnp.maximum(acc, 0.0) if relu else acc

    return pl.pallas_call(
        body,
        grid=(N // BN,),
        in_specs=[pl.BlockSpec((BN, F), lambda i: (i, 0)),
                  pl.BlockSpec((BN, FH), lambda i: (i, 0)),
                  pl.BlockSpec((BN, FH), lambda i: (i, 0)),
                  pl.BlockSpec((2, 2, BN, FH), lambda i: (0, 0, i, 0)),
                  pl.BlockSpec((3, F, F), lambda i: (0, 0, 0)),
                  pl.BlockSpec((1, F), lambda i: (0, 0))],
        out_specs=pl.BlockSpec((BN, F), lambda i: (i, 0)),
        out_shape=jax.ShapeDtypeStruct((N, F), jnp.float32),
    )(h, t10, t11, r, W, b2d)


def _pool_tc(batf, h3, fcW, fcb2d):
    nblk = N // BN

    def body(bt_ref, h_ref, fw_ref, fb_ref, o_ref, pacc, cacc):
        i = pl.program_id(0)

        @pl.when(i == 0)
        def _():
            pacc[...] = jnp.zeros_like(pacc)
            cacc[...] = jnp.zeros_like(cacc)

        bb = bt_ref[0]
        gi = lax.broadcasted_iota(jnp.int32, (G, 1), 0).astype(jnp.float32)
        oh = jnp.where(bb == gi, 1.0, 0.0)
        pacc[...] += jnp.dot(oh, h_ref[...], preferred_element_type=jnp.float32)
        cacc[...] += jnp.broadcast_to(
            jnp.sum(oh, axis=1, keepdims=True), (G, F))

        @pl.when(i == nblk - 1)
        def _():
            pooled = pacc[...] / jnp.maximum(cacc[...], 1.0)
            o_ref[...] = jnp.dot(pooled, fw_ref[...],
                                 preferred_element_type=jnp.float32) + fb_ref[...]

    return pl.pallas_call(
        body,
        grid=(nblk,),
        in_specs=[pl.BlockSpec((1, 1, BN), lambda i: (i, 0, 0)),
                  pl.BlockSpec((BN, F), lambda i: (i, 0)),
                  pl.BlockSpec((F, F), lambda i: (0, 0)),
                  pl.BlockSpec((1, F), lambda i: (0, 0))],
        out_specs=pl.BlockSpec((G, F), lambda i: (0, 0)),
        out_shape=jax.ShapeDtypeStruct((G, F), jnp.float32),
        scratch_shapes=[pltpu.VMEM((G, F), jnp.float32),
                        pltpu.VMEM((G, F), jnp.float32)],
    )(batf, h3, fcW, fcb2d)


def kernel(x, edge_index, edge_weight, batch, W1, b1, W2, b2, W3, b3, fcW, fcb):
    srcR = edge_index[0].reshape(NROWS, R)
    dstR = edge_index[1].reshape(NROWS, R)
    wR = edge_weight.reshape(NROWS, R)
    zN = jnp.zeros((N,), jnp.float32)
    zNH = jnp.zeros((N, FH), jnp.float32)
    normR = _prep(srcR, dstR, wR, zN)
    batf = batch.astype(jnp.float32).reshape(N // BN, 1, BN)

    h = x
    for (W, b, relu) in ((W1, b1, True), (W2, b2, True), (W3, b3, False)):
        q = _spmm(h[:, :FH], h[:, FH:], srcR, dstR, normR, zNH)
        t10, t11 = _sum_tc(q)
        r = _spmm(t10, t11, srcR, dstR, normR, zNH)
        h = _combine_tc(h, t10, t11, r, W, b.reshape(1, F), relu)

    return _pool_tc(batf, h, fcW, fcb.reshape(1, F))
